# baseline (speedup 1.0000x reference)
import jax
import jax.numpy as jnp
from jax import lax
from jax.experimental import pallas as pl
from jax.experimental.pallas import tpu as pltpu
from jax.experimental.pallas import tpu_sc as plsc

_N = 100000
_D = 128
_S = 1024
_NPAD = 100352
_NSTREAM = 16
_TBLK = 896
_CHUNK = 7168
_NFULL = 13
_TAIL = 6816


def _matvec_body(*refs):
    o_ref = refs[-1]
    w_ref = refs[-3]
    b = refs[-2][0, 0]
    for k, xr in enumerate(refs[:_NSTREAM]):
        yt = lax.dot_general(
            w_ref[...], xr[...], (((1,), (1,)), ((), ())),
            preferred_element_type=jnp.float32,
        )
        o_ref[pl.ds(k * _TBLK, _TBLK)] = (yt + b).reshape(_TBLK)


def _segment_body(y_hbm, idx_hbm, out_hbm, yv, iv, yv2, iv2, zv, acc, sem):
    c = lax.axis_index("c")
    s = lax.axis_index("s")

    @pl.when(jnp.logical_and(c == 0, s < _NFULL))
    def _stage():
        pltpu.async_copy(idx_hbm.at[pl.ds(s * _CHUNK, _CHUNK)], iv, sem)
        pltpu.async_copy(y_hbm.at[pl.ds(s * _CHUNK, _CHUNK)], yv, sem)

    @pl.when(jnp.logical_and(c == 0, s == _NFULL))
    def _stage_tail():
        pltpu.async_copy(idx_hbm.at[pl.ds(_NFULL * _CHUNK, _TAIL)], iv2, sem)
        pltpu.async_copy(y_hbm.at[pl.ds(_NFULL * _CHUNK, _TAIL)], yv2, sem)

    @pl.when(jnp.logical_and(c == 0, s == 0))
    def _zero():
        for i in range(_S // 16):
            zv[pl.ds(i * 16, 16)] = jnp.zeros((16,), jnp.float32)
        pltpu.sync_copy(zv, acc)

    plsc.subcore_barrier()

    @pl.when(jnp.logical_and(c == 0, s < _NFULL))
    def _scatter():
        pltpu.make_async_copy(idx_hbm.at[pl.ds(s * _CHUNK, _CHUNK)], iv, sem).wait()
        pltpu.make_async_copy(y_hbm.at[pl.ds(s * _CHUNK, _CHUNK)], yv, sem).wait()
        pltpu.sync_copy(yv, acc.at[iv], add=True)

    @pl.when(jnp.logical_and(c == 0, s == _NFULL))
    def _scatter_tail():
        pltpu.make_async_copy(idx_hbm.at[pl.ds(_NFULL * _CHUNK, _TAIL)], iv2, sem).wait()
        pltpu.make_async_copy(y_hbm.at[pl.ds(_NFULL * _CHUNK, _TAIL)], yv2, sem).wait()
        pltpu.sync_copy(yv2, acc.at[iv2], add=True)

    plsc.subcore_barrier()

    @pl.when(jnp.logical_and(c == 0, s == 0))
    def _writeback():
        pltpu.sync_copy(acc, out_hbm)


def kernel(node_embedding, batch, Wout, bout):
    w = Wout.astype(jnp.float32).reshape(1, _D)
    b2 = bout.reshape(1, 1).astype(jnp.float32)
    y = pl.pallas_call(
        _matvec_body,
        grid=(_NPAD // (_NSTREAM * _TBLK),),
        in_specs=[
            pl.BlockSpec((_TBLK, _D), lambda i, k=k: (_NSTREAM * i + k, 0))
            for k in range(_NSTREAM)
        ] + [
            pl.BlockSpec((1, _D), lambda i: (0, 0)),
            pl.BlockSpec((1, 1), lambda i: (0, 0), memory_space=pltpu.SMEM),
        ],
        out_specs=pl.BlockSpec((_NSTREAM * _TBLK,), lambda i: (i,)),
        out_shape=jax.ShapeDtypeStruct((_NPAD,), jnp.float32),
    )(*([node_embedding] * _NSTREAM), w, b2)

    seg = pl.kernel(
        _segment_body,
        out_type=jax.ShapeDtypeStruct((_S,), jnp.float32),
        mesh=plsc.VectorSubcoreMesh(core_axis_name="c", subcore_axis_name="s"),
        scratch_types=[
            pltpu.VMEM((_CHUNK,), jnp.float32),
            pltpu.VMEM((_CHUNK,), jnp.int32),
            pltpu.VMEM((_TAIL,), jnp.float32),
            pltpu.VMEM((_TAIL,), jnp.int32),
            pltpu.VMEM((_S,), jnp.float32),
            pltpu.VMEM_SHARED((_S,), jnp.float32),
            pltpu.SemaphoreType.DMA,
        ],
    )(y, batch.astype(jnp.int32))
    return seg


# speedup vs baseline: 1.0304x; 1.0304x over previous
import jax
import jax.numpy as jnp
from jax import lax
from jax.experimental import pallas as pl
from jax.experimental.pallas import tpu as pltpu
from jax.experimental.pallas import tpu_sc as plsc

_N = 100000
_D = 128
_S = 1024
_NPAD = 100352
_NSTREAM = 8
_TBLK = 1792
_CHUNK = 7168
_NFULL = 13
_TAIL = 6816
_K = 4
_ACC = _K * _S


def _matvec_body(*refs):
    o_ref = refs[-1]
    w_ref = refs[-3]
    b = refs[-2][0, 0]
    for k, xr in enumerate(refs[:_NSTREAM]):
        yt = lax.dot_general(
            w_ref[...], xr[...], (((1,), (1,)), ((), ())),
            preferred_element_type=jnp.float32,
        )
        o_ref[pl.ds(k * _TBLK, _TBLK)] = (yt + b).reshape(_TBLK)


def _segment_body(y_hbm, idx_hbm, out_hbm, yv, iv, yv2, iv2, zv, av, acc, sem):
    c = lax.axis_index("c")
    s = lax.axis_index("s")

    @pl.when(jnp.logical_and(c == 0, s < _NFULL))
    def _stage():
        pltpu.async_copy(idx_hbm.at[pl.ds(s * _CHUNK, _CHUNK)], iv, sem)
        pltpu.async_copy(y_hbm.at[pl.ds(s * _CHUNK, _CHUNK)], yv, sem)

    @pl.when(jnp.logical_and(c == 0, s == _NFULL))
    def _stage_tail():
        pltpu.async_copy(idx_hbm.at[pl.ds(_NFULL * _CHUNK, _TAIL)], iv2, sem)
        pltpu.async_copy(y_hbm.at[pl.ds(_NFULL * _CHUNK, _TAIL)], yv2, sem)

    @pl.when(jnp.logical_and(c == 0, s == 0))
    def _zero():
        for i in range(_S // 16):
            zv[pl.ds(i * 16, 16)] = jnp.zeros((16,), jnp.float32)
        for r in range(_K):
            pltpu.sync_copy(zv, acc.at[pl.ds(r * _S, _S)])

    plsc.subcore_barrier()

    @pl.when(jnp.logical_and(c == 0, s < _NFULL))
    def _scatter():
        pltpu.make_async_copy(idx_hbm.at[pl.ds(s * _CHUNK, _CHUNK)], iv, sem).wait()
        pltpu.make_async_copy(y_hbm.at[pl.ds(s * _CHUNK, _CHUNK)], yv, sem).wait()
        pltpu.sync_copy(yv, acc.at[iv], add=True)

    @pl.when(jnp.logical_and(c == 0, s == _NFULL))
    def _scatter_tail():
        pltpu.make_async_copy(idx_hbm.at[pl.ds(_NFULL * _CHUNK, _TAIL)], iv2, sem).wait()
        pltpu.make_async_copy(y_hbm.at[pl.ds(_NFULL * _CHUNK, _TAIL)], yv2, sem).wait()
        pltpu.sync_copy(yv2, acc.at[iv2], add=True)

    plsc.subcore_barrier()

    @pl.when(jnp.logical_and(c == 0, s == 0))
    def _writeback():
        pltpu.sync_copy(acc, av)

        def _comb(i, carry):
            off = pl.multiple_of(i * 16, 16)
            tot = av[pl.ds(off, 16)]
            for r in range(1, _K):
                tot = tot + av[pl.ds(r * _S + off, 16)]
            zv[pl.ds(off, 16)] = tot
            return carry

        lax.fori_loop(0, _S // 16, _comb, 0)
        pltpu.sync_copy(zv, out_hbm)


def kernel(node_embedding, batch, Wout, bout):
    w = Wout.astype(jnp.float32).reshape(1, _D)
    b2 = bout.reshape(1, 1).astype(jnp.float32)
    y = pl.pallas_call(
        _matvec_body,
        grid=(_NPAD // (_NSTREAM * _TBLK),),
        in_specs=[
            pl.BlockSpec((_TBLK, _D), lambda i, k=k: (_NSTREAM * i + k, 0))
            for k in range(_NSTREAM)
        ] + [
            pl.BlockSpec((1, _D), lambda i: (0, 0)),
            pl.BlockSpec((1, 1), lambda i: (0, 0), memory_space=pltpu.SMEM),
        ],
        out_specs=pl.BlockSpec((_NSTREAM * _TBLK,), lambda i: (i,)),
        out_shape=jax.ShapeDtypeStruct((_NPAD,), jnp.float32),
    )(*([node_embedding] * _NSTREAM), w, b2)

    seg = pl.kernel(
        _segment_body,
        out_type=jax.ShapeDtypeStruct((_S,), jnp.float32),
        mesh=plsc.VectorSubcoreMesh(core_axis_name="c", subcore_axis_name="s"),
        scratch_types=[
            pltpu.VMEM((_CHUNK,), jnp.float32),
            pltpu.VMEM((_CHUNK,), jnp.int32),
            pltpu.VMEM((_TAIL,), jnp.float32),
            pltpu.VMEM((_TAIL,), jnp.int32),
            pltpu.VMEM((_S,), jnp.float32),
            pltpu.VMEM((_ACC,), jnp.float32),
            pltpu.VMEM_SHARED((_ACC,), jnp.float32),
            pltpu.SemaphoreType.DMA,
        ],
    )(y, batch.astype(jnp.int32) + _S * (jnp.arange(_N, dtype=jnp.int32) & (_K - 1)))
    return seg
